# baseline (speedup 1.0000x reference)
import functools

import jax
import jax.numpy as jnp
from jax import lax
from jax.experimental import pallas as pl
from jax.experimental.pallas import tpu as pltpu
from jax.experimental.pallas import tpu_sc as plsc

N_ENT = 100000
N_REL = 1000
D = 128
B_POS = 16384
B_NEG = 65536
B_ALL = B_POS + B_NEG

NC = 2
NS = 16
NW = NC * NS
CHUNK = 128

N_STRIPE = 4
TRI_S = B_ALL // N_STRIPE
ROWS_B = 512
DW = D // 2


def _sc_gather(combo_ti, entity_residual, has_img_f, combo_rel,
               eids3d, rids3d):
    ne_chunks = eids3d.shape[1]
    nr_chunks = rids3d.shape[1]
    n_eid = NW * ne_chunks * CHUNK
    n_rid = NW * nr_chunks * CHUNK
    mesh = plsc.VectorSubcoreMesh(core_axis_name="c", subcore_axis_name="s")

    @functools.partial(
        pl.kernel,
        mesh=mesh,
        out_type=[
            jax.ShapeDtypeStruct((n_eid, D), jnp.int32),
            jax.ShapeDtypeStruct((n_eid, D), jnp.float32),
            jax.ShapeDtypeStruct((n_eid,), jnp.float32),
            jax.ShapeDtypeStruct((n_rid, D), jnp.int32),
        ],
        scratch_types=[
            pltpu.VMEM((ne_chunks, CHUNK), jnp.int32),
            pltpu.VMEM((nr_chunks, CHUNK), jnp.int32),
            pltpu.VMEM((2, CHUNK, D), jnp.int32),
            pltpu.VMEM((2, CHUNK, D), jnp.float32),
            pltpu.VMEM((2, CHUNK), jnp.float32),
            pltpu.SemaphoreType.DMA,
            pltpu.SemaphoreType.DMA,
            pltpu.SemaphoreType.DMA,
            pltpu.SemaphoreType.DMA,
        ],
    )
    def k(ti_h, res_h, mask_h, rel_h, eids_h, rids_h,
          out_ti, out_r, out_m, out_rel,
          idx2d, rid2d, ib, fb, mb, sg0, sg1, ss0, ss1):
        wid = lax.axis_index("c") * NS + lax.axis_index("s")
        sg = (sg0, sg1)
        ss = (ss0, ss1)

        pltpu.sync_copy(eids_h.at[wid], idx2d)
        pltpu.sync_copy(rids_h.at[wid], rid2d)

        egat = ((ti_h, ib, out_ti), (res_h, fb, out_r), (mask_h, mb, out_m))
        rgat = ((rel_h, ib, out_rel),)

        def fire_gather(gats, idx, b):
            for src, buf, _ in gats:
                pltpu.async_copy(src.at[idx], buf.at[b], sg[b])

        def wait_gather(gats, b):
            for src, buf, _ in gats:
                pltpu.make_async_copy(src.at[pl.ds(0, CHUNK)], buf.at[b],
                                      sg[b]).wait()

        def fire_store(gats, off, b):
            for _, buf, out in gats:
                pltpu.async_copy(buf.at[b], out.at[pl.ds(off, CHUNK)], ss[b])

        def wait_store(gats, b):
            for _, buf, out in gats:
                pltpu.make_async_copy(buf.at[b], out.at[pl.ds(0, CHUNK)],
                                      ss[b]).wait()

        def run(gats, idxref, nchunks, base):
            def step(c, b):
                @pl.when(c >= 2)
                def _():
                    wait_store(gats, b)

                fire_gather(gats, idxref.at[c], b)

                @pl.when(c >= 1)
                def _():
                    wait_gather(gats, 1 - b)
                    fire_store(gats, base + (c - 1) * CHUNK, 1 - b)

            def body(c2, _):
                for b in (0, 1):
                    step(c2 * 2 + b, b)
                return 0

            lax.fori_loop(0, nchunks // 2, body, 0)
            if nchunks % 2:
                step(nchunks - 1, (nchunks - 1) % 2)
            last = (nchunks - 1) % 2
            wait_gather(gats, last)
            fire_store(gats, base + (nchunks - 1) * CHUNK, last)
            wait_store(gats, 1 - last)
            wait_store(gats, last)

        run(egat, idx2d, ne_chunks, wid * ne_chunks * CHUNK)
        run(rgat, rid2d, nr_chunks, wid * nr_chunks * CHUNK)

    return k(combo_ti, entity_residual, has_img_f, combo_rel, eids3d, rids3d)


def _softplus(x):
    return jnp.maximum(x, 0.0) + jnp.log1p(jnp.exp(-jnp.abs(x)))


def _unpack(w):
    lo = jax.lax.bitcast_convert_type(w << 16, jnp.float32)
    hi = jax.lax.bitcast_convert_type(w & jnp.int32(-65536), jnp.float32)
    return jnp.concatenate([lo, hi], axis=1)


def _fuse_side(t, v_raw, m, resid, rf, Wg, bg, gamma, beta, scale, v_missing):
    v = jnp.where(m > 0.5, v_raw, v_missing)
    x = jnp.concatenate([t, v, rf], axis=1)
    g = jax.nn.sigmoid(
        jnp.dot(x, Wg, preferred_element_type=jnp.float32) + bg)
    z = g * t + (1.0 - g) * v
    mu = jnp.mean(z, axis=-1, keepdims=True)
    zc = z - mu
    var = jnp.mean(zc * zc, axis=-1, keepdims=True)
    z = zc * jax.lax.rsqrt(var + 1e-05) * gamma + beta
    return z + scale * resid


def _make_fuse_kernel(pos_blks, with_scale_term):
    def _fuse_kernel(tih_ref, tit_ref, mh_ref, mt_ref,
                     rh_ref, rt_ref, rel_ref,
                     wg_ref, bg_ref, gm_ref, bt_ref, vm_ref, rs_ref,
                     out_ref):
        i = pl.program_id(0)
        rs = rs_ref[0, 0]
        scale = _softplus(rs)
        bg = bg_ref[...]
        gm = gm_ref[...]
        bt = bt_ref[...]
        vm = vm_ref[...]
        wg = wg_ref[...]

        tih = tih_ref[...]
        tit = tit_ref[...]
        rel = rel_ref[...]
        rff = _unpack(rel[:, :DW])
        zh = _fuse_side(_unpack(tih[:, :DW]), _unpack(tih[:, DW:]),
                        mh_ref[...], rh_ref[...],
                        rff, wg, bg, gm, bt, scale, vm)
        zt = _fuse_side(_unpack(tit[:, :DW]), _unpack(tit[:, DW:]),
                        mt_ref[...], rt_ref[...],
                        rff, wg, bg, gm, bt, scale, vm)
        rd = _unpack(rel[:, DW:])

        hr, hi = zh[:, :D // 2], zh[:, D // 2:]
        rr, ri = rd[:, :D // 2], rd[:, D // 2:]
        tr, ti = zt[:, :D // 2], zt[:, D // 2:]
        s = jnp.sum(hr * (rr * tr + ri * ti) + hi * (rr * ti - ri * tr),
                    axis=1)

        if pos_blks == 0:
            contrib = jnp.sum(_softplus(s)) / B_NEG
        else:
            contrib = jnp.where(
                i < pos_blks,
                jnp.sum(_softplus(-s)) / B_POS,
                jnp.sum(_softplus(s)) / B_NEG,
            )

        @pl.when(i == 0)
        def _():
            first = contrib
            if with_scale_term:
                first = first + 1e-04 * scale * scale
            out_ref[...] = jnp.reshape(first, (1, 1))

        @pl.when(i > 0)
        def _():
            out_ref[...] += jnp.reshape(contrib, (1, 1))

    return _fuse_kernel


def _tc_fuse(gat_ti, gat_m2, gat_r, gat_rel,
             Wg, bg, gamma, beta, v_missing, rscale,
             n_trip, pos_blks, with_scale_term):
    n_blk = n_trip // ROWS_B
    row_spec_h = pl.BlockSpec((ROWS_B, D), lambda i: (i, 0))
    row_spec_t = pl.BlockSpec((ROWS_B, D), lambda i: (i + n_blk, 0))
    m_spec_h = pl.BlockSpec((ROWS_B, 1), lambda i: (i, 0))
    m_spec_t = pl.BlockSpec((ROWS_B, 1), lambda i: (i + n_blk, 0))
    rel_spec = pl.BlockSpec((ROWS_B, D), lambda i: (i, 0))

    def p_spec(shape):
        return pl.BlockSpec(shape, lambda i: (0, 0))

    return pl.pallas_call(
        _make_fuse_kernel(pos_blks, with_scale_term),
        grid=(n_blk,),
        in_specs=[
            row_spec_h, row_spec_t,
            m_spec_h, m_spec_t,
            row_spec_h, row_spec_t,
            rel_spec,
            p_spec((3 * D, D)),
            p_spec((1, D)), p_spec((1, D)), p_spec((1, D)), p_spec((1, D)),
            p_spec((1, 1)),
        ],
        out_specs=pl.BlockSpec((1, 1), lambda i: (0, 0)),
        out_shape=jax.ShapeDtypeStruct((1, 1), jnp.float32),
        compiler_params=pltpu.CompilerParams(
            dimension_semantics=("arbitrary",)),
    )(gat_ti, gat_ti, gat_m2, gat_m2, gat_r, gat_r,
      gat_rel, Wg, bg, gamma, beta, v_missing, rscale)


L2_ROWS = 1000
L2_BLKS = N_ENT // L2_ROWS


def _l2_kernel(er_ref, out_ref):
    i = pl.program_id(0)
    x = er_ref[...]
    part = jnp.sum(x * x) * (1e-06 / (N_ENT * D))

    @pl.when(i == 0)
    def _():
        out_ref[...] = jnp.reshape(part, (1, 1))

    @pl.when(i > 0)
    def _():
        out_ref[...] += jnp.reshape(part, (1, 1))


def _tc_l2(entity_residual):
    return pl.pallas_call(
        _l2_kernel,
        grid=(L2_BLKS,),
        in_specs=[pl.BlockSpec((L2_ROWS, D), lambda i: (i, 0))],
        out_specs=pl.BlockSpec((1, 1), lambda i: (0, 0)),
        out_shape=jax.ShapeDtypeStruct((1, 1), jnp.float32),
        compiler_params=pltpu.CompilerParams(
            dimension_semantics=("arbitrary",)),
    )(entity_residual)


def kernel(text_emb, img_emb, v_missing, entity_residual, residual_scale,
           rel_emb_fusion, Wg, bg, gamma, beta, rel_emb_dec, has_img,
           pos_triples, neg_triples):
    heids = jnp.concatenate([pos_triples[:, 0], neg_triples[:, 0]])
    teids = jnp.concatenate([pos_triples[:, 2], neg_triples[:, 2]])
    rids = jnp.concatenate([pos_triples[:, 1], neg_triples[:, 1]])
    has_img_f = has_img.astype(jnp.float32)

    perm = jnp.arange(D).reshape(2, DW).T.reshape(D)

    def pack(tbl):
        b = tbl.astype(jnp.bfloat16)[:, perm]
        return jax.lax.bitcast_convert_type(b.reshape(-1, DW, 2), jnp.int32)

    combo_ti = jnp.concatenate([pack(text_emb), pack(img_emb)], axis=1)
    combo_rel = jnp.concatenate([pack(rel_emb_fusion), pack(rel_emb_dec)],
                                axis=1)

    bg2 = bg.reshape(1, D)
    gamma2 = gamma.reshape(1, D)
    beta2 = beta.reshape(1, D)
    vm2 = v_missing.reshape(1, D)
    rs2 = jnp.asarray(residual_scale, jnp.float32).reshape(1, 1)

    l2 = _tc_l2(entity_residual)

    total = l2[0, 0]
    bces = []
    for s in range(N_STRIPE):
        lo, hi = s * TRI_S, (s + 1) * TRI_S
        eids_s = jnp.concatenate([heids[lo:hi], teids[lo:hi]])
        rids_s = rids[lo:hi]
        n_pos_s = min(max(B_POS - lo, 0), TRI_S)
        assert n_pos_s % ROWS_B == 0

        if s >= 2:
            eids_s, _ = lax.optimization_barrier((eids_s, bces[s - 2]))

        gat_ti, gat_r, gat_m, gat_rel = _sc_gather(
            combo_ti, entity_residual, has_img_f, combo_rel,
            eids_s.reshape(NW, 2 * TRI_S // (NW * CHUNK), CHUNK),
            rids_s.reshape(NW, TRI_S // (NW * CHUNK), CHUNK))

        bce_s = _tc_fuse(
            gat_ti, gat_m.reshape(2 * TRI_S, 1), gat_r, gat_rel,
            Wg, bg2, gamma2, beta2, vm2, rs2,
            TRI_S, n_pos_s // ROWS_B, s == 0)
        bces.append(bce_s)
        total = total + bce_s[0, 0]

    return total


# speedup vs baseline: 2.1414x; 2.1414x over previous
import functools

import jax
import jax.numpy as jnp
from jax import lax
from jax.experimental import pallas as pl
from jax.experimental.pallas import tpu as pltpu
from jax.experimental.pallas import tpu_sc as plsc

N_ENT = 100000
N_REL = 1000
D = 128
B_POS = 16384
B_NEG = 65536
B_ALL = B_POS + B_NEG

NC = 2
NS = 16
NW = NC * NS
CHUNK = 128

N_STRIPE = 4
TRI_S = B_ALL // N_STRIPE
ROWS_B = 512
DW = D // 2


def _sc_gather(combo_ti, entity_residual, has_img_f, combo_rel,
               eids3d, rids3d):
    ne_chunks = eids3d.shape[1]
    nr_chunks = rids3d.shape[1]
    n_eid = NW * ne_chunks * CHUNK
    n_rid = NW * nr_chunks * CHUNK
    mesh = plsc.VectorSubcoreMesh(core_axis_name="c", subcore_axis_name="s")

    @functools.partial(
        pl.kernel,
        mesh=mesh,
        out_type=[
            jax.ShapeDtypeStruct((n_eid, D), jnp.int32),
            jax.ShapeDtypeStruct((n_eid, D), jnp.float32),
            jax.ShapeDtypeStruct((n_eid,), jnp.float32),
            jax.ShapeDtypeStruct((n_rid, D), jnp.int32),
        ],
        scratch_types=[
            pltpu.VMEM((ne_chunks, CHUNK), jnp.int32),
            pltpu.VMEM((nr_chunks, CHUNK), jnp.int32),
            pltpu.VMEM((2, CHUNK, D), jnp.int32),
            pltpu.VMEM((2, CHUNK, D), jnp.float32),
            pltpu.VMEM((2, CHUNK), jnp.float32),
            pltpu.SemaphoreType.DMA,
            pltpu.SemaphoreType.DMA,
            pltpu.SemaphoreType.DMA,
            pltpu.SemaphoreType.DMA,
        ],
    )
    def k(ti_h, res_h, mask_h, rel_h, eids_h, rids_h,
          out_ti, out_r, out_m, out_rel,
          idx2d, rid2d, ib, fb, mb, sg0, sg1, ss0, ss1):
        wid = lax.axis_index("c") * NS + lax.axis_index("s")
        sg = (sg0, sg1)
        ss = (ss0, ss1)

        pltpu.sync_copy(eids_h.at[wid], idx2d)
        pltpu.sync_copy(rids_h.at[wid], rid2d)

        egat = ((ti_h, ib, out_ti), (res_h, fb, out_r), (mask_h, mb, out_m))
        rgat = ((rel_h, ib, out_rel),)

        def fire_gather(gats, idx, b):
            for src, buf, _ in gats:
                pltpu.async_copy(src.at[idx], buf.at[b], sg[b])

        def wait_gather(gats, b):
            for src, buf, _ in gats:
                pltpu.make_async_copy(src.at[pl.ds(0, CHUNK)], buf.at[b],
                                      sg[b]).wait()

        def fire_store(gats, off, b):
            for _, buf, out in gats:
                pltpu.async_copy(buf.at[b], out.at[pl.ds(off, CHUNK)], ss[b])

        def wait_store(gats, b):
            for _, buf, out in gats:
                pltpu.make_async_copy(buf.at[b], out.at[pl.ds(0, CHUNK)],
                                      ss[b]).wait()

        def run(gats, idxref, nchunks, base):
            def step(c, b):
                @pl.when(c >= 2)
                def _():
                    wait_store(gats, b)

                fire_gather(gats, idxref.at[c], b)

                @pl.when(c >= 1)
                def _():
                    wait_gather(gats, 1 - b)
                    fire_store(gats, base + (c - 1) * CHUNK, 1 - b)

            def body(c2, _):
                for b in (0, 1):
                    step(c2 * 2 + b, b)
                return 0

            lax.fori_loop(0, nchunks // 2, body, 0)
            if nchunks % 2:
                step(nchunks - 1, (nchunks - 1) % 2)
            last = (nchunks - 1) % 2
            wait_gather(gats, last)
            fire_store(gats, base + (nchunks - 1) * CHUNK, last)
            wait_store(gats, 1 - last)
            wait_store(gats, last)

        run(egat, idx2d, ne_chunks, wid * ne_chunks * CHUNK)
        run(rgat, rid2d, nr_chunks, wid * nr_chunks * CHUNK)

    return k(combo_ti, entity_residual, has_img_f, combo_rel, eids3d, rids3d)


def _softplus(x):
    return jnp.maximum(x, 0.0) + jnp.log1p(jnp.exp(-jnp.abs(x)))


def _unpack(w):
    lo = jax.lax.bitcast_convert_type(w << 16, jnp.float32)
    hi = jax.lax.bitcast_convert_type(w & jnp.int32(-65536), jnp.float32)
    return jnp.concatenate([lo, hi], axis=1)


def _fuse_side(t, v_raw, m, resid, rf, Wg, bg, gamma, beta, scale, v_missing):
    v = jnp.where(m > 0.5, v_raw, v_missing)
    x = jnp.concatenate([t, v, rf], axis=1)
    g = jax.nn.sigmoid(
        jnp.dot(x, Wg, preferred_element_type=jnp.float32) + bg)
    z = g * t + (1.0 - g) * v
    mu = jnp.mean(z, axis=-1, keepdims=True)
    zc = z - mu
    var = jnp.mean(zc * zc, axis=-1, keepdims=True)
    z = zc * jax.lax.rsqrt(var + 1e-05) * gamma + beta
    return z + scale * resid


def _make_fuse_kernel(pos_blks, with_scale_term):
    def _fuse_kernel(tih_ref, tit_ref, mh_ref, mt_ref,
                     rh_ref, rt_ref, rel_ref,
                     wg_ref, bg_ref, gm_ref, bt_ref, vm_ref, rs_ref,
                     out_ref):
        i = pl.program_id(0)
        rs = rs_ref[0, 0]
        scale = _softplus(rs)
        bg = bg_ref[...]
        gm = gm_ref[...]
        bt = bt_ref[...]
        vm = vm_ref[...]
        wg = wg_ref[...]

        tih = tih_ref[...]
        tit = tit_ref[...]
        rel = rel_ref[...]
        rff = _unpack(rel[:, :DW])
        zh = _fuse_side(_unpack(tih[:, :DW]), _unpack(tih[:, DW:]),
                        mh_ref[...], rh_ref[...],
                        rff, wg, bg, gm, bt, scale, vm)
        zt = _fuse_side(_unpack(tit[:, :DW]), _unpack(tit[:, DW:]),
                        mt_ref[...], rt_ref[...],
                        rff, wg, bg, gm, bt, scale, vm)
        rd = _unpack(rel[:, DW:])

        hr, hi = zh[:, :D // 2], zh[:, D // 2:]
        rr, ri = rd[:, :D // 2], rd[:, D // 2:]
        tr, ti = zt[:, :D // 2], zt[:, D // 2:]
        s = jnp.sum(hr * (rr * tr + ri * ti) + hi * (rr * ti - ri * tr),
                    axis=1)

        if pos_blks == 0:
            contrib = jnp.sum(_softplus(s)) / B_NEG
        else:
            contrib = jnp.where(
                i < pos_blks,
                jnp.sum(_softplus(-s)) / B_POS,
                jnp.sum(_softplus(s)) / B_NEG,
            )

        @pl.when(i == 0)
        def _():
            first = contrib
            if with_scale_term:
                first = first + 1e-04 * scale * scale
            out_ref[...] = jnp.reshape(first, (1, 1))

        @pl.when(i > 0)
        def _():
            out_ref[...] += jnp.reshape(contrib, (1, 1))

    return _fuse_kernel


def _tc_fuse(gat_ti, gat_m2, gat_r, gat_rel,
             Wg, bg, gamma, beta, v_missing, rscale,
             n_trip, pos_blks, with_scale_term):
    n_blk = n_trip // ROWS_B
    row_spec_h = pl.BlockSpec((ROWS_B, D), lambda i: (i, 0))
    row_spec_t = pl.BlockSpec((ROWS_B, D), lambda i: (i + n_blk, 0))
    m_spec_h = pl.BlockSpec((ROWS_B, 1), lambda i: (i, 0))
    m_spec_t = pl.BlockSpec((ROWS_B, 1), lambda i: (i + n_blk, 0))
    rel_spec = pl.BlockSpec((ROWS_B, D), lambda i: (i, 0))

    def p_spec(shape):
        return pl.BlockSpec(shape, lambda i: (0, 0))

    return pl.pallas_call(
        _make_fuse_kernel(pos_blks, with_scale_term),
        grid=(n_blk,),
        in_specs=[
            row_spec_h, row_spec_t,
            m_spec_h, m_spec_t,
            row_spec_h, row_spec_t,
            rel_spec,
            p_spec((3 * D, D)),
            p_spec((1, D)), p_spec((1, D)), p_spec((1, D)), p_spec((1, D)),
            p_spec((1, 1)),
        ],
        out_specs=pl.BlockSpec((1, 1), lambda i: (0, 0)),
        out_shape=jax.ShapeDtypeStruct((1, 1), jnp.float32),
        compiler_params=pltpu.CompilerParams(
            dimension_semantics=("arbitrary",)),
    )(gat_ti, gat_ti, gat_m2, gat_m2, gat_r, gat_r,
      gat_rel, Wg, bg, gamma, beta, v_missing, rscale)


def _pack_half(x):
    lo = jax.lax.bitcast_convert_type(x[:, :DW], jnp.int32)
    hi = jax.lax.bitcast_convert_type(x[:, DW:], jnp.int32)

    def rnd(b):
        return b + jnp.int32(0x7FFF) + ((b >> 16) & jnp.int32(1))

    lo16 = (rnd(lo) >> 16) & jnp.int32(0xFFFF)
    hi16 = rnd(hi) & jnp.int32(-65536)
    return hi16 | lo16


def _pack_kernel(a_ref, b_ref, out_ref):
    out_ref[...] = jnp.concatenate(
        [_pack_half(a_ref[...]), _pack_half(b_ref[...])], axis=1)


def _tc_pack(a, b):
    n = a.shape[0]
    rows = 1000
    spec = pl.BlockSpec((rows, D), lambda i: (i, 0))
    return pl.pallas_call(
        _pack_kernel,
        grid=(n // rows,),
        in_specs=[spec, spec],
        out_specs=spec,
        out_shape=jax.ShapeDtypeStruct((n, D), jnp.int32),
    )(a, b)


L2_ROWS = 1000
L2_BLKS = N_ENT // L2_ROWS


def _l2_kernel(er_ref, out_ref):
    i = pl.program_id(0)
    x = er_ref[...]
    part = jnp.sum(x * x) * (1e-06 / (N_ENT * D))

    @pl.when(i == 0)
    def _():
        out_ref[...] = jnp.reshape(part, (1, 1))

    @pl.when(i > 0)
    def _():
        out_ref[...] += jnp.reshape(part, (1, 1))


def _tc_l2(entity_residual):
    return pl.pallas_call(
        _l2_kernel,
        grid=(L2_BLKS,),
        in_specs=[pl.BlockSpec((L2_ROWS, D), lambda i: (i, 0))],
        out_specs=pl.BlockSpec((1, 1), lambda i: (0, 0)),
        out_shape=jax.ShapeDtypeStruct((1, 1), jnp.float32),
        compiler_params=pltpu.CompilerParams(
            dimension_semantics=("arbitrary",)),
    )(entity_residual)


def kernel(text_emb, img_emb, v_missing, entity_residual, residual_scale,
           rel_emb_fusion, Wg, bg, gamma, beta, rel_emb_dec, has_img,
           pos_triples, neg_triples):
    heids = jnp.concatenate([pos_triples[:, 0], neg_triples[:, 0]])
    teids = jnp.concatenate([pos_triples[:, 2], neg_triples[:, 2]])
    rids = jnp.concatenate([pos_triples[:, 1], neg_triples[:, 1]])
    has_img_f = has_img.astype(jnp.float32)

    combo_ti = _tc_pack(text_emb, img_emb)
    combo_rel = _tc_pack(rel_emb_fusion, rel_emb_dec)

    bg2 = bg.reshape(1, D)
    gamma2 = gamma.reshape(1, D)
    beta2 = beta.reshape(1, D)
    vm2 = v_missing.reshape(1, D)
    rs2 = jnp.asarray(residual_scale, jnp.float32).reshape(1, 1)

    l2 = _tc_l2(entity_residual)

    total = l2[0, 0]
    bces = []
    for s in range(N_STRIPE):
        lo, hi = s * TRI_S, (s + 1) * TRI_S
        eids_s = jnp.concatenate([heids[lo:hi], teids[lo:hi]])
        rids_s = rids[lo:hi]
        n_pos_s = min(max(B_POS - lo, 0), TRI_S)
        assert n_pos_s % ROWS_B == 0

        if s >= 2:
            eids_s, _ = lax.optimization_barrier((eids_s, bces[s - 2]))

        gat_ti, gat_r, gat_m, gat_rel = _sc_gather(
            combo_ti, entity_residual, has_img_f, combo_rel,
            eids_s.reshape(NW, 2 * TRI_S // (NW * CHUNK), CHUNK),
            rids_s.reshape(NW, TRI_S // (NW * CHUNK), CHUNK))

        bce_s = _tc_fuse(
            gat_ti, gat_m.reshape(2 * TRI_S, 1), gat_r, gat_rel,
            Wg, bg2, gamma2, beta2, vm2, rs2,
            TRI_S, n_pos_s // ROWS_B, s == 0)
        bces.append(bce_s)
        total = total + bce_s[0, 0]

    return total


# speedup vs baseline: 2.8297x; 1.3215x over previous
import functools

import jax
import jax.numpy as jnp
from jax import lax
from jax.experimental import pallas as pl
from jax.experimental.pallas import tpu as pltpu
from jax.experimental.pallas import tpu_sc as plsc

N_ENT = 100000
N_REL = 1000
D = 128
B_POS = 16384
B_NEG = 65536
B_ALL = B_POS + B_NEG

NC = 2
NS = 16
NW = NC * NS
CHUNK = 128

N_STRIPE = 4
TRI_S = B_ALL // N_STRIPE
ROWS_B = 512
DW = D // 2


def _sc_gather(combo_ti, entity_residual, has_img_f, combo_rel,
               eids3d, rids3d):
    ne_chunks = eids3d.shape[1]
    nr_chunks = rids3d.shape[1]
    n_eid = NW * ne_chunks * CHUNK
    n_rid = NW * nr_chunks * CHUNK
    mesh = plsc.VectorSubcoreMesh(core_axis_name="c", subcore_axis_name="s")

    @functools.partial(
        pl.kernel,
        mesh=mesh,
        out_type=[
            jax.ShapeDtypeStruct((n_eid, D), jnp.int32),
            jax.ShapeDtypeStruct((n_eid, D), jnp.float32),
            jax.ShapeDtypeStruct((n_eid,), jnp.float32),
            jax.ShapeDtypeStruct((n_rid, D), jnp.int32),
        ],
        scratch_types=[
            pltpu.VMEM((ne_chunks, CHUNK), jnp.int32),
            pltpu.VMEM((nr_chunks, CHUNK), jnp.int32),
            pltpu.VMEM((2, CHUNK, D), jnp.int32),
            pltpu.VMEM((2, CHUNK, D), jnp.float32),
            pltpu.VMEM((2, CHUNK), jnp.float32),
            pltpu.SemaphoreType.DMA,
            pltpu.SemaphoreType.DMA,
            pltpu.SemaphoreType.DMA,
            pltpu.SemaphoreType.DMA,
        ],
    )
    def k(ti_h, res_h, mask_h, rel_h, eids_h, rids_h,
          out_ti, out_r, out_m, out_rel,
          idx2d, rid2d, ib, fb, mb, sg0, sg1, ss0, ss1):
        wid = lax.axis_index("c") * NS + lax.axis_index("s")
        sg = (sg0, sg1)
        ss = (ss0, ss1)

        pltpu.sync_copy(eids_h.at[wid], idx2d)
        pltpu.sync_copy(rids_h.at[wid], rid2d)

        egat = ((ti_h, ib, out_ti), (res_h, fb, out_r), (mask_h, mb, out_m))
        rgat = ((rel_h, ib, out_rel),)

        def fire_gather(gats, idx, b):
            for src, buf, _ in gats:
                pltpu.async_copy(src.at[idx], buf.at[b], sg[b])

        def wait_gather(gats, b):
            for src, buf, _ in gats:
                pltpu.make_async_copy(src.at[pl.ds(0, CHUNK)], buf.at[b],
                                      sg[b]).wait()

        def fire_store(gats, off, b):
            for _, buf, out in gats:
                pltpu.async_copy(buf.at[b], out.at[pl.ds(off, CHUNK)], ss[b])

        def wait_store(gats, b):
            for _, buf, out in gats:
                pltpu.make_async_copy(buf.at[b], out.at[pl.ds(0, CHUNK)],
                                      ss[b]).wait()

        def run(gats, idxref, nchunks, base):
            def step(c, b):
                @pl.when(c >= 2)
                def _():
                    wait_store(gats, b)

                fire_gather(gats, idxref.at[c], b)

                @pl.when(c >= 1)
                def _():
                    wait_gather(gats, 1 - b)
                    fire_store(gats, base + (c - 1) * CHUNK, 1 - b)

            def body(c2, _):
                for b in (0, 1):
                    step(c2 * 2 + b, b)
                return 0

            lax.fori_loop(0, nchunks // 2, body, 0)
            if nchunks % 2:
                step(nchunks - 1, (nchunks - 1) % 2)
            last = (nchunks - 1) % 2
            wait_gather(gats, last)
            fire_store(gats, base + (nchunks - 1) * CHUNK, last)
            wait_store(gats, 1 - last)
            wait_store(gats, last)

        run(egat, idx2d, ne_chunks, wid * ne_chunks * CHUNK)
        run(rgat, rid2d, nr_chunks, wid * nr_chunks * CHUNK)

    return k(combo_ti, entity_residual, has_img_f, combo_rel, eids3d, rids3d)


def _softplus(x):
    return jnp.maximum(x, 0.0) + jnp.log1p(jnp.exp(-jnp.abs(x)))


def _unpack_lo(w):
    return jax.lax.bitcast_convert_type(w << 16, jnp.float32)


def _unpack_hi(w):
    return jax.lax.bitcast_convert_type(w & jnp.int32(-65536), jnp.float32)


def _fuse_side(t, v_raw, m, resid, rf, Wg, bg, gamma, beta, scale, v_missing):
    v = jnp.where(m > 0.5, v_raw, v_missing)
    x = jnp.concatenate([t, v, rf], axis=1)
    g = jax.nn.sigmoid(
        jnp.dot(x, Wg, preferred_element_type=jnp.float32) + bg)
    z = g * t + (1.0 - g) * v
    mu = jnp.mean(z, axis=-1, keepdims=True)
    zc = z - mu
    var = jnp.mean(zc * zc, axis=-1, keepdims=True)
    z = zc * jax.lax.rsqrt(var + 1e-05) * gamma + beta
    return z + scale * resid


def _make_fuse_kernel(pos_blks, with_scale_term):
    def _fuse_kernel(tih_ref, tit_ref, mh_ref, mt_ref,
                     rh_ref, rt_ref, rel_ref,
                     wg_ref, bg_ref, gm_ref, bt_ref, vm_ref, rs_ref,
                     out_ref):
        i = pl.program_id(0)
        rs = rs_ref[0, 0]
        scale = _softplus(rs)
        bg = bg_ref[...]
        gm = gm_ref[...]
        bt = bt_ref[...]
        vm = vm_ref[...]
        wg = wg_ref[...]

        tih = tih_ref[...]
        tit = tit_ref[...]
        rel = rel_ref[...]
        rff = _unpack_lo(rel)
        zh = _fuse_side(_unpack_lo(tih), _unpack_hi(tih),
                        mh_ref[...], rh_ref[...],
                        rff, wg, bg, gm, bt, scale, vm)
        zt = _fuse_side(_unpack_lo(tit), _unpack_hi(tit),
                        mt_ref[...], rt_ref[...],
                        rff, wg, bg, gm, bt, scale, vm)
        rd = _unpack_hi(rel)

        hr, hi = zh[:, :D // 2], zh[:, D // 2:]
        rr, ri = rd[:, :D // 2], rd[:, D // 2:]
        tr, ti = zt[:, :D // 2], zt[:, D // 2:]
        s = jnp.sum(hr * (rr * tr + ri * ti) + hi * (rr * ti - ri * tr),
                    axis=1)

        if pos_blks == 0:
            contrib = jnp.sum(_softplus(s)) / B_NEG
        else:
            contrib = jnp.where(
                i < pos_blks,
                jnp.sum(_softplus(-s)) / B_POS,
                jnp.sum(_softplus(s)) / B_NEG,
            )

        @pl.when(i == 0)
        def _():
            first = contrib
            if with_scale_term:
                first = first + 1e-04 * scale * scale
            out_ref[...] = jnp.reshape(first, (1, 1))

        @pl.when(i > 0)
        def _():
            out_ref[...] += jnp.reshape(contrib, (1, 1))

    return _fuse_kernel


def _tc_fuse(gat_ti, gat_m2, gat_r, gat_rel,
             Wg, bg, gamma, beta, v_missing, rscale,
             n_trip, pos_blks, with_scale_term):
    n_blk = n_trip // ROWS_B
    row_spec_h = pl.BlockSpec((ROWS_B, D), lambda i: (i, 0))
    row_spec_t = pl.BlockSpec((ROWS_B, D), lambda i: (i + n_blk, 0))
    m_spec_h = pl.BlockSpec((ROWS_B, 1), lambda i: (i, 0))
    m_spec_t = pl.BlockSpec((ROWS_B, 1), lambda i: (i + n_blk, 0))
    rel_spec = pl.BlockSpec((ROWS_B, D), lambda i: (i, 0))

    def p_spec(shape):
        return pl.BlockSpec(shape, lambda i: (0, 0))

    return pl.pallas_call(
        _make_fuse_kernel(pos_blks, with_scale_term),
        grid=(n_blk,),
        in_specs=[
            row_spec_h, row_spec_t,
            m_spec_h, m_spec_t,
            row_spec_h, row_spec_t,
            rel_spec,
            p_spec((3 * D, D)),
            p_spec((1, D)), p_spec((1, D)), p_spec((1, D)), p_spec((1, D)),
            p_spec((1, 1)),
        ],
        out_specs=pl.BlockSpec((1, 1), lambda i: (0, 0)),
        out_shape=jax.ShapeDtypeStruct((1, 1), jnp.float32),
        compiler_params=pltpu.CompilerParams(
            dimension_semantics=("arbitrary",)),
    )(gat_ti, gat_ti, gat_m2, gat_m2, gat_r, gat_r,
      gat_rel, Wg, bg, gamma, beta, v_missing, rscale)


def _pack_kernel(a_ref, b_ref, out_ref):
    a = jax.lax.bitcast_convert_type(a_ref[...], jnp.int32)
    b = jax.lax.bitcast_convert_type(b_ref[...], jnp.int32)

    def rnd(x):
        return x + jnp.int32(0x7FFF) + ((x >> 16) & jnp.int32(1))

    out_ref[...] = ((rnd(a) >> 16) & jnp.int32(0xFFFF)) | \
        (rnd(b) & jnp.int32(-65536))


def _tc_pack(a, b):
    n = a.shape[0]
    rows = 1000
    spec = pl.BlockSpec((rows, D), lambda i: (i, 0))
    return pl.pallas_call(
        _pack_kernel,
        grid=(n // rows,),
        in_specs=[spec, spec],
        out_specs=spec,
        out_shape=jax.ShapeDtypeStruct((n, D), jnp.int32),
    )(a, b)


L2_ROWS = 1000
L2_BLKS = N_ENT // L2_ROWS


def _l2_kernel(er_ref, out_ref):
    i = pl.program_id(0)
    x = er_ref[...]
    part = jnp.sum(x * x) * (1e-06 / (N_ENT * D))

    @pl.when(i == 0)
    def _():
        out_ref[...] = jnp.reshape(part, (1, 1))

    @pl.when(i > 0)
    def _():
        out_ref[...] += jnp.reshape(part, (1, 1))


def _tc_l2(entity_residual):
    return pl.pallas_call(
        _l2_kernel,
        grid=(L2_BLKS,),
        in_specs=[pl.BlockSpec((L2_ROWS, D), lambda i: (i, 0))],
        out_specs=pl.BlockSpec((1, 1), lambda i: (0, 0)),
        out_shape=jax.ShapeDtypeStruct((1, 1), jnp.float32),
        compiler_params=pltpu.CompilerParams(
            dimension_semantics=("arbitrary",)),
    )(entity_residual)


def kernel(text_emb, img_emb, v_missing, entity_residual, residual_scale,
           rel_emb_fusion, Wg, bg, gamma, beta, rel_emb_dec, has_img,
           pos_triples, neg_triples):
    heids = jnp.concatenate([pos_triples[:, 0], neg_triples[:, 0]])
    teids = jnp.concatenate([pos_triples[:, 2], neg_triples[:, 2]])
    rids = jnp.concatenate([pos_triples[:, 1], neg_triples[:, 1]])
    has_img_f = has_img.astype(jnp.float32)

    combo_ti = _tc_pack(text_emb, img_emb)
    combo_rel = _tc_pack(rel_emb_fusion, rel_emb_dec)

    bg2 = bg.reshape(1, D)
    gamma2 = gamma.reshape(1, D)
    beta2 = beta.reshape(1, D)
    vm2 = v_missing.reshape(1, D)
    rs2 = jnp.asarray(residual_scale, jnp.float32).reshape(1, 1)

    l2 = _tc_l2(entity_residual)

    total = l2[0, 0]
    bces = []
    for s in range(N_STRIPE):
        lo, hi = s * TRI_S, (s + 1) * TRI_S
        eids_s = jnp.concatenate([heids[lo:hi], teids[lo:hi]])
        rids_s = rids[lo:hi]
        n_pos_s = min(max(B_POS - lo, 0), TRI_S)
        assert n_pos_s % ROWS_B == 0

        if s >= 2:
            eids_s, _ = lax.optimization_barrier((eids_s, bces[s - 2]))

        gat_ti, gat_r, gat_m, gat_rel = _sc_gather(
            combo_ti, entity_residual, has_img_f, combo_rel,
            eids_s.reshape(NW, 2 * TRI_S // (NW * CHUNK), CHUNK),
            rids_s.reshape(NW, TRI_S // (NW * CHUNK), CHUNK))

        bce_s = _tc_fuse(
            gat_ti, gat_m.reshape(2 * TRI_S, 1), gat_r, gat_rel,
            Wg, bg2, gamma2, beta2, vm2, rs2,
            TRI_S, n_pos_s // ROWS_B, s == 0)
        bces.append(bce_s)
        total = total + bce_s[0, 0]

    return total


# speedup vs baseline: 3.2539x; 1.1499x over previous
import functools

import jax
import jax.numpy as jnp
from jax import lax
from jax.experimental import pallas as pl
from jax.experimental.pallas import tpu as pltpu
from jax.experimental.pallas import tpu_sc as plsc

N_ENT = 100000
N_REL = 1000
D = 128
B_POS = 16384
B_NEG = 65536
B_ALL = B_POS + B_NEG

NC = 2
NS = 16
NW = NC * NS
CHUNK = 128

N_STRIPE = 4
TRI_S = B_ALL // N_STRIPE
ROWS_B = 512
DW = D // 2


def _sc_gather(combo_ti, entity_residual, combo_rel, eids3d, rids3d):
    ne_chunks = eids3d.shape[1]
    nr_chunks = rids3d.shape[1]
    n_eid = NW * ne_chunks * CHUNK
    n_rid = NW * nr_chunks * CHUNK
    mesh = plsc.VectorSubcoreMesh(core_axis_name="c", subcore_axis_name="s")

    @functools.partial(
        pl.kernel,
        mesh=mesh,
        out_type=[
            jax.ShapeDtypeStruct((n_eid, D), jnp.int32),
            jax.ShapeDtypeStruct((n_eid, D), jnp.float32),
            jax.ShapeDtypeStruct((n_rid, D), jnp.int32),
        ],
        scratch_types=[
            pltpu.VMEM((ne_chunks, CHUNK), jnp.int32),
            pltpu.VMEM((nr_chunks, CHUNK), jnp.int32),
            pltpu.VMEM((2, CHUNK, D), jnp.int32),
            pltpu.VMEM((2, CHUNK, D), jnp.float32),
            pltpu.SemaphoreType.DMA,
            pltpu.SemaphoreType.DMA,
            pltpu.SemaphoreType.DMA,
            pltpu.SemaphoreType.DMA,
        ],
    )
    def k(ti_h, res_h, rel_h, eids_h, rids_h,
          out_ti, out_r, out_rel,
          idx2d, rid2d, ib, fb, sg0, sg1, ss0, ss1):
        wid = lax.axis_index("c") * NS + lax.axis_index("s")
        sg = (sg0, sg1)
        ss = (ss0, ss1)

        pltpu.sync_copy(eids_h.at[wid], idx2d)
        pltpu.sync_copy(rids_h.at[wid], rid2d)

        egat = ((ti_h, ib, out_ti), (res_h, fb, out_r))
        rgat = ((rel_h, ib, out_rel),)

        def fire_gather(gats, idx, b):
            for src, buf, _ in gats:
                pltpu.async_copy(src.at[idx], buf.at[b], sg[b])

        def wait_gather(gats, b):
            for src, buf, _ in gats:
                pltpu.make_async_copy(src.at[pl.ds(0, CHUNK)], buf.at[b],
                                      sg[b]).wait()

        def fire_store(gats, off, b):
            for _, buf, out in gats:
                pltpu.async_copy(buf.at[b], out.at[pl.ds(off, CHUNK)], ss[b])

        def wait_store(gats, b):
            for _, buf, out in gats:
                pltpu.make_async_copy(buf.at[b], out.at[pl.ds(0, CHUNK)],
                                      ss[b]).wait()

        def run(gats, idxref, nchunks, base):
            def step(c, b):
                @pl.when(c >= 2)
                def _():
                    wait_store(gats, b)

                fire_gather(gats, idxref.at[c], b)

                @pl.when(c >= 1)
                def _():
                    wait_gather(gats, 1 - b)
                    fire_store(gats, base + (c - 1) * CHUNK, 1 - b)

            def body(c2, _):
                for b in (0, 1):
                    step(c2 * 2 + b, b)
                return 0

            lax.fori_loop(0, nchunks // 2, body, 0)
            if nchunks % 2:
                step(nchunks - 1, (nchunks - 1) % 2)
            last = (nchunks - 1) % 2
            wait_gather(gats, last)
            fire_store(gats, base + (nchunks - 1) * CHUNK, last)
            wait_store(gats, 1 - last)
            wait_store(gats, last)

        run(egat, idx2d, ne_chunks, wid * ne_chunks * CHUNK)
        run(rgat, rid2d, nr_chunks, wid * nr_chunks * CHUNK)

    return k(combo_ti, entity_residual, combo_rel, eids3d, rids3d)


def _softplus(x):
    return jnp.maximum(x, 0.0) + jnp.log1p(jnp.exp(-jnp.abs(x)))


def _unpack_lo(w):
    return jax.lax.bitcast_convert_type(w << 16, jnp.float32)


def _unpack_hi(w):
    return jax.lax.bitcast_convert_type(w & jnp.int32(-65536), jnp.float32)


def _fuse_side(t, v, resid, rf, Wg, bg, gamma, beta, scale):
    x = jnp.concatenate([t, v, rf], axis=1)
    g = jax.nn.sigmoid(
        jnp.dot(x, Wg, preferred_element_type=jnp.float32) + bg)
    z = g * t + (1.0 - g) * v
    mu = jnp.mean(z, axis=-1, keepdims=True)
    zc = z - mu
    var = jnp.mean(zc * zc, axis=-1, keepdims=True)
    z = zc * jax.lax.rsqrt(var + 1e-05) * gamma + beta
    return z + scale * resid


def _make_fuse_kernel(pos_blks, with_scale_term):
    def _fuse_kernel(tih_ref, tit_ref, rh_ref, rt_ref, rel_ref,
                     wg_ref, bg_ref, gm_ref, bt_ref, rs_ref,
                     out_ref):
        i = pl.program_id(0)
        rs = rs_ref[0, 0]
        scale = _softplus(rs)
        bg = bg_ref[...]
        gm = gm_ref[...]
        bt = bt_ref[...]
        wg = wg_ref[...]

        tih = tih_ref[...]
        tit = tit_ref[...]
        rel = rel_ref[...]
        rff = _unpack_lo(rel)
        zh = _fuse_side(_unpack_lo(tih), _unpack_hi(tih), rh_ref[...],
                        rff, wg, bg, gm, bt, scale)
        zt = _fuse_side(_unpack_lo(tit), _unpack_hi(tit), rt_ref[...],
                        rff, wg, bg, gm, bt, scale)
        rd = _unpack_hi(rel)

        hr, hi = zh[:, :D // 2], zh[:, D // 2:]
        rr, ri = rd[:, :D // 2], rd[:, D // 2:]
        tr, ti = zt[:, :D // 2], zt[:, D // 2:]
        s = jnp.sum(hr * (rr * tr + ri * ti) + hi * (rr * ti - ri * tr),
                    axis=1)

        if pos_blks == 0:
            contrib = jnp.sum(_softplus(s)) / B_NEG
        else:
            contrib = jnp.where(
                i < pos_blks,
                jnp.sum(_softplus(-s)) / B_POS,
                jnp.sum(_softplus(s)) / B_NEG,
            )

        @pl.when(i == 0)
        def _():
            first = contrib
            if with_scale_term:
                first = first + 1e-04 * scale * scale
            out_ref[...] = jnp.reshape(first, (1, 1))

        @pl.when(i > 0)
        def _():
            out_ref[...] += jnp.reshape(contrib, (1, 1))

    return _fuse_kernel


def _tc_fuse(gat_ti, gat_r, gat_rel,
             Wg, bg, gamma, beta, rscale,
             n_trip, pos_blks, with_scale_term):
    n_blk = n_trip // ROWS_B
    row_spec_h = pl.BlockSpec((ROWS_B, D), lambda i: (i, 0))
    row_spec_t = pl.BlockSpec((ROWS_B, D), lambda i: (i + n_blk, 0))
    rel_spec = pl.BlockSpec((ROWS_B, D), lambda i: (i, 0))

    def p_spec(shape):
        return pl.BlockSpec(shape, lambda i: (0, 0))

    return pl.pallas_call(
        _make_fuse_kernel(pos_blks, with_scale_term),
        grid=(n_blk,),
        in_specs=[
            row_spec_h, row_spec_t,
            row_spec_h, row_spec_t,
            rel_spec,
            p_spec((3 * D, D)),
            p_spec((1, D)), p_spec((1, D)), p_spec((1, D)),
            p_spec((1, 1)),
        ],
        out_specs=pl.BlockSpec((1, 1), lambda i: (0, 0)),
        out_shape=jax.ShapeDtypeStruct((1, 1), jnp.float32),
        compiler_params=pltpu.CompilerParams(
            dimension_semantics=("arbitrary",)),
    )(gat_ti, gat_ti, gat_r, gat_r,
      gat_rel, Wg, bg, gamma, beta, rscale)


def _pack_words(a, b):
    ai = jax.lax.bitcast_convert_type(a, jnp.int32)
    bi = jax.lax.bitcast_convert_type(b, jnp.int32)

    def rnd(x):
        return x + jnp.int32(0x7FFF) + ((x >> 16) & jnp.int32(1))

    return ((rnd(ai) >> 16) & jnp.int32(0xFFFF)) | (rnd(bi) & jnp.int32(-65536))


def _pack_kernel(a_ref, b_ref, out_ref):
    out_ref[...] = _pack_words(a_ref[...], b_ref[...])


def _tc_pack(a, b):
    n = a.shape[0]
    rows = min(n, 1000)
    spec = pl.BlockSpec((rows, D), lambda i: (i, 0))
    return pl.pallas_call(
        _pack_kernel,
        grid=(n // rows,),
        in_specs=[spec, spec],
        out_specs=spec,
        out_shape=jax.ShapeDtypeStruct((n, D), jnp.int32),
    )(a, b)


def _pack_ent_kernel(t_ref, v_ref, m_ref, vm_ref, er_ref, out_ref, l2_ref):
    i = pl.program_id(0)
    v = jnp.where(m_ref[...] > 0.5, v_ref[...], vm_ref[...])
    out_ref[...] = _pack_words(t_ref[...], v)
    x = er_ref[...]
    part = jnp.sum(x * x) * (1e-06 / (N_ENT * D))

    @pl.when(i == 0)
    def _():
        l2_ref[...] = jnp.reshape(part, (1, 1))

    @pl.when(i > 0)
    def _():
        l2_ref[...] += jnp.reshape(part, (1, 1))


def _tc_pack_ent(text_emb, img_emb, has2d, vm2, entity_residual):
    rows = 1000
    spec = pl.BlockSpec((rows, D), lambda i: (i, 0))
    return pl.pallas_call(
        _pack_ent_kernel,
        grid=(N_ENT // rows,),
        in_specs=[spec, spec,
                  pl.BlockSpec((rows, 1), lambda i: (i, 0)),
                  pl.BlockSpec((1, D), lambda i: (0, 0)),
                  spec],
        out_specs=[spec, pl.BlockSpec((1, 1), lambda i: (0, 0))],
        out_shape=[jax.ShapeDtypeStruct((N_ENT, D), jnp.int32),
                   jax.ShapeDtypeStruct((1, 1), jnp.float32)],
        compiler_params=pltpu.CompilerParams(
            dimension_semantics=("arbitrary",)),
    )(text_emb, img_emb, has2d, vm2, entity_residual)


L2_ROWS = 1000
L2_BLKS = N_ENT // L2_ROWS


def _l2_kernel(er_ref, out_ref):
    i = pl.program_id(0)
    x = er_ref[...]
    part = jnp.sum(x * x) * (1e-06 / (N_ENT * D))

    @pl.when(i == 0)
    def _():
        out_ref[...] = jnp.reshape(part, (1, 1))

    @pl.when(i > 0)
    def _():
        out_ref[...] += jnp.reshape(part, (1, 1))


def _tc_l2(entity_residual):
    return pl.pallas_call(
        _l2_kernel,
        grid=(L2_BLKS,),
        in_specs=[pl.BlockSpec((L2_ROWS, D), lambda i: (i, 0))],
        out_specs=pl.BlockSpec((1, 1), lambda i: (0, 0)),
        out_shape=jax.ShapeDtypeStruct((1, 1), jnp.float32),
        compiler_params=pltpu.CompilerParams(
            dimension_semantics=("arbitrary",)),
    )(entity_residual)


def kernel(text_emb, img_emb, v_missing, entity_residual, residual_scale,
           rel_emb_fusion, Wg, bg, gamma, beta, rel_emb_dec, has_img,
           pos_triples, neg_triples):
    heids = jnp.concatenate([pos_triples[:, 0], neg_triples[:, 0]])
    teids = jnp.concatenate([pos_triples[:, 2], neg_triples[:, 2]])
    rids = jnp.concatenate([pos_triples[:, 1], neg_triples[:, 1]])
    bg2 = bg.reshape(1, D)
    gamma2 = gamma.reshape(1, D)
    beta2 = beta.reshape(1, D)
    vm2 = v_missing.reshape(1, D)
    rs2 = jnp.asarray(residual_scale, jnp.float32).reshape(1, 1)
    has2d = has_img.astype(jnp.float32).reshape(N_ENT, 1)

    combo_ti, l2 = _tc_pack_ent(text_emb, img_emb, has2d, vm2,
                                entity_residual)
    combo_rel = _tc_pack(rel_emb_fusion, rel_emb_dec)

    total = l2[0, 0]
    bces = []
    for s in range(N_STRIPE):
        lo, hi = s * TRI_S, (s + 1) * TRI_S
        eids_s = jnp.concatenate([heids[lo:hi], teids[lo:hi]])
        rids_s = rids[lo:hi]
        n_pos_s = min(max(B_POS - lo, 0), TRI_S)
        assert n_pos_s % ROWS_B == 0

        if s >= 2:
            eids_s, _ = lax.optimization_barrier((eids_s, bces[s - 2]))

        gat_ti, gat_r, gat_rel = _sc_gather(
            combo_ti, entity_residual, combo_rel,
            eids_s.reshape(NW, 2 * TRI_S // (NW * CHUNK), CHUNK),
            rids_s.reshape(NW, TRI_S // (NW * CHUNK), CHUNK))

        bce_s = _tc_fuse(
            gat_ti, gat_r, gat_rel,
            Wg, bg2, gamma2, beta2, rs2,
            TRI_S, n_pos_s // ROWS_B, s == 0)
        bces.append(bce_s)
        total = total + bce_s[0, 0]

    return total


# speedup vs baseline: 3.2843x; 1.0094x over previous
import functools

import jax
import jax.numpy as jnp
from jax import lax
from jax.experimental import pallas as pl
from jax.experimental.pallas import tpu as pltpu
from jax.experimental.pallas import tpu_sc as plsc

N_ENT = 100000
N_REL = 1000
D = 128
B_POS = 16384
B_NEG = 65536
B_ALL = B_POS + B_NEG

NC = 2
NS = 16
NW = NC * NS
CHUNK = 128

N_STRIPE = 5
TRI_S = B_ALL // N_STRIPE
ROWS_B = 512
DW = D // 2


def _sc_gather(combo_ti, entity_residual, combo_rel, eids3d, rids3d):
    ne_chunks = eids3d.shape[1]
    nr_chunks = rids3d.shape[1]
    n_eid = NW * ne_chunks * CHUNK
    n_rid = NW * nr_chunks * CHUNK
    mesh = plsc.VectorSubcoreMesh(core_axis_name="c", subcore_axis_name="s")

    @functools.partial(
        pl.kernel,
        mesh=mesh,
        out_type=[
            jax.ShapeDtypeStruct((n_eid, D), jnp.int32),
            jax.ShapeDtypeStruct((n_eid, D), jnp.float32),
            jax.ShapeDtypeStruct((n_rid, D), jnp.int32),
        ],
        scratch_types=[
            pltpu.VMEM((ne_chunks, CHUNK), jnp.int32),
            pltpu.VMEM((nr_chunks, CHUNK), jnp.int32),
            pltpu.VMEM((2, CHUNK, D), jnp.int32),
            pltpu.VMEM((2, CHUNK, D), jnp.float32),
            pltpu.SemaphoreType.DMA,
            pltpu.SemaphoreType.DMA,
            pltpu.SemaphoreType.DMA,
            pltpu.SemaphoreType.DMA,
        ],
    )
    def k(ti_h, res_h, rel_h, eids_h, rids_h,
          out_ti, out_r, out_rel,
          idx2d, rid2d, ib, fb, sg0, sg1, ss0, ss1):
        wid = lax.axis_index("c") * NS + lax.axis_index("s")
        sg = (sg0, sg1)
        ss = (ss0, ss1)

        pltpu.sync_copy(eids_h.at[wid], idx2d)
        pltpu.sync_copy(rids_h.at[wid], rid2d)

        egat = ((ti_h, ib, out_ti), (res_h, fb, out_r))
        rgat = ((rel_h, ib, out_rel),)

        def fire_gather(gats, idx, b):
            for src, buf, _ in gats:
                pltpu.async_copy(src.at[idx], buf.at[b], sg[b])

        def wait_gather(gats, b):
            for src, buf, _ in gats:
                pltpu.make_async_copy(src.at[pl.ds(0, CHUNK)], buf.at[b],
                                      sg[b]).wait()

        def fire_store(gats, off, b):
            for _, buf, out in gats:
                pltpu.async_copy(buf.at[b], out.at[pl.ds(off, CHUNK)], ss[b])

        def wait_store(gats, b):
            for _, buf, out in gats:
                pltpu.make_async_copy(buf.at[b], out.at[pl.ds(0, CHUNK)],
                                      ss[b]).wait()

        def run(gats, idxref, nchunks, base):
            def step(c, b):
                @pl.when(c >= 2)
                def _():
                    wait_store(gats, b)

                fire_gather(gats, idxref.at[c], b)

                @pl.when(c >= 1)
                def _():
                    wait_gather(gats, 1 - b)
                    fire_store(gats, base + (c - 1) * CHUNK, 1 - b)

            def body(c2, _):
                for b in (0, 1):
                    step(c2 * 2 + b, b)
                return 0

            lax.fori_loop(0, nchunks // 2, body, 0)
            if nchunks % 2:
                step(nchunks - 1, (nchunks - 1) % 2)
            last = (nchunks - 1) % 2
            wait_gather(gats, last)
            fire_store(gats, base + (nchunks - 1) * CHUNK, last)
            wait_store(gats, 1 - last)
            wait_store(gats, last)

        run(egat, idx2d, ne_chunks, wid * ne_chunks * CHUNK)
        run(rgat, rid2d, nr_chunks, wid * nr_chunks * CHUNK)

    return k(combo_ti, entity_residual, combo_rel, eids3d, rids3d)


def _softplus(x):
    return jnp.maximum(x, 0.0) + jnp.log1p(jnp.exp(-jnp.abs(x)))


def _unpack_lo(w):
    return jax.lax.bitcast_convert_type(w << 16, jnp.float32)


def _unpack_hi(w):
    return jax.lax.bitcast_convert_type(w & jnp.int32(-65536), jnp.float32)


def _fuse_side(t, v, resid, rf, Wg, bg, gamma, beta, scale):
    x = jnp.concatenate([t, v, rf], axis=1)
    g = jax.nn.sigmoid(
        jnp.dot(x, Wg, preferred_element_type=jnp.float32) + bg)
    z = g * t + (1.0 - g) * v
    mu = jnp.mean(z, axis=-1, keepdims=True)
    zc = z - mu
    var = jnp.mean(zc * zc, axis=-1, keepdims=True)
    z = zc * jax.lax.rsqrt(var + 1e-05) * gamma + beta
    return z + scale * resid


def _make_fuse_kernel(pos_blks, with_scale_term):
    def _fuse_kernel(tih_ref, tit_ref, rh_ref, rt_ref, rel_ref,
                     wg_ref, bg_ref, gm_ref, bt_ref, rs_ref,
                     out_ref):
        i = pl.program_id(0)
        rs = rs_ref[0, 0]
        scale = _softplus(rs)
        bg = bg_ref[...]
        gm = gm_ref[...]
        bt = bt_ref[...]
        wg = wg_ref[...]

        tih = tih_ref[...]
        tit = tit_ref[...]
        rel = rel_ref[...]
        rff = _unpack_lo(rel)
        zh = _fuse_side(_unpack_lo(tih), _unpack_hi(tih), rh_ref[...],
                        rff, wg, bg, gm, bt, scale)
        zt = _fuse_side(_unpack_lo(tit), _unpack_hi(tit), rt_ref[...],
                        rff, wg, bg, gm, bt, scale)
        rd = _unpack_hi(rel)

        hr, hi = zh[:, :D // 2], zh[:, D // 2:]
        rr, ri = rd[:, :D // 2], rd[:, D // 2:]
        tr, ti = zt[:, :D // 2], zt[:, D // 2:]
        s = jnp.sum(hr * (rr * tr + ri * ti) + hi * (rr * ti - ri * tr),
                    axis=1)

        if pos_blks == 0:
            contrib = jnp.sum(_softplus(s)) / B_NEG
        else:
            contrib = jnp.where(
                i < pos_blks,
                jnp.sum(_softplus(-s)) / B_POS,
                jnp.sum(_softplus(s)) / B_NEG,
            )

        @pl.when(i == 0)
        def _():
            first = contrib
            if with_scale_term:
                first = first + 1e-04 * scale * scale
            out_ref[...] = jnp.reshape(first, (1, 1))

        @pl.when(i > 0)
        def _():
            out_ref[...] += jnp.reshape(contrib, (1, 1))

    return _fuse_kernel


def _tc_fuse(gat_ti, gat_r, gat_rel,
             Wg, bg, gamma, beta, rscale,
             n_trip, pos_blks, with_scale_term):
    n_blk = n_trip // ROWS_B
    row_spec_h = pl.BlockSpec((ROWS_B, D), lambda i: (i, 0))
    row_spec_t = pl.BlockSpec((ROWS_B, D), lambda i: (i + n_blk, 0))
    rel_spec = pl.BlockSpec((ROWS_B, D), lambda i: (i, 0))

    def p_spec(shape):
        return pl.BlockSpec(shape, lambda i: (0, 0))

    return pl.pallas_call(
        _make_fuse_kernel(pos_blks, with_scale_term),
        grid=(n_blk,),
        in_specs=[
            row_spec_h, row_spec_t,
            row_spec_h, row_spec_t,
            rel_spec,
            p_spec((3 * D, D)),
            p_spec((1, D)), p_spec((1, D)), p_spec((1, D)),
            p_spec((1, 1)),
        ],
        out_specs=pl.BlockSpec((1, 1), lambda i: (0, 0)),
        out_shape=jax.ShapeDtypeStruct((1, 1), jnp.float32),
        compiler_params=pltpu.CompilerParams(
            dimension_semantics=("arbitrary",)),
    )(gat_ti, gat_ti, gat_r, gat_r,
      gat_rel, Wg, bg, gamma, beta, rscale)


def _pack_words(a, b):
    ai = jax.lax.bitcast_convert_type(a, jnp.int32)
    bi = jax.lax.bitcast_convert_type(b, jnp.int32)

    def rnd(x):
        return x + jnp.int32(0x7FFF) + ((x >> 16) & jnp.int32(1))

    return ((rnd(ai) >> 16) & jnp.int32(0xFFFF)) | (rnd(bi) & jnp.int32(-65536))


def _pack_kernel(a_ref, b_ref, out_ref):
    out_ref[...] = _pack_words(a_ref[...], b_ref[...])


def _tc_pack(a, b):
    n = a.shape[0]
    rows = min(n, 1000)
    spec = pl.BlockSpec((rows, D), lambda i: (i, 0))
    return pl.pallas_call(
        _pack_kernel,
        grid=(n // rows,),
        in_specs=[spec, spec],
        out_specs=spec,
        out_shape=jax.ShapeDtypeStruct((n, D), jnp.int32),
    )(a, b)


def _pack_ent_kernel(t_ref, v_ref, m_ref, vm_ref, er_ref, out_ref, l2_ref):
    i = pl.program_id(0)
    v = jnp.where(m_ref[...] > 0.5, v_ref[...], vm_ref[...])
    out_ref[...] = _pack_words(t_ref[...], v)
    x = er_ref[...]
    part = jnp.sum(x * x) * (1e-06 / (N_ENT * D))

    @pl.when(i == 0)
    def _():
        l2_ref[...] = jnp.reshape(part, (1, 1))

    @pl.when(i > 0)
    def _():
        l2_ref[...] += jnp.reshape(part, (1, 1))


def _tc_pack_ent(text_emb, img_emb, has2d, vm2, entity_residual):
    rows = 1000
    spec = pl.BlockSpec((rows, D), lambda i: (i, 0))
    return pl.pallas_call(
        _pack_ent_kernel,
        grid=(N_ENT // rows,),
        in_specs=[spec, spec,
                  pl.BlockSpec((rows, 1), lambda i: (i, 0)),
                  pl.BlockSpec((1, D), lambda i: (0, 0)),
                  spec],
        out_specs=[spec, pl.BlockSpec((1, 1), lambda i: (0, 0))],
        out_shape=[jax.ShapeDtypeStruct((N_ENT, D), jnp.int32),
                   jax.ShapeDtypeStruct((1, 1), jnp.float32)],
        compiler_params=pltpu.CompilerParams(
            dimension_semantics=("arbitrary",)),
    )(text_emb, img_emb, has2d, vm2, entity_residual)


L2_ROWS = 1000
L2_BLKS = N_ENT // L2_ROWS


def _l2_kernel(er_ref, out_ref):
    i = pl.program_id(0)
    x = er_ref[...]
    part = jnp.sum(x * x) * (1e-06 / (N_ENT * D))

    @pl.when(i == 0)
    def _():
        out_ref[...] = jnp.reshape(part, (1, 1))

    @pl.when(i > 0)
    def _():
        out_ref[...] += jnp.reshape(part, (1, 1))


def _tc_l2(entity_residual):
    return pl.pallas_call(
        _l2_kernel,
        grid=(L2_BLKS,),
        in_specs=[pl.BlockSpec((L2_ROWS, D), lambda i: (i, 0))],
        out_specs=pl.BlockSpec((1, 1), lambda i: (0, 0)),
        out_shape=jax.ShapeDtypeStruct((1, 1), jnp.float32),
        compiler_params=pltpu.CompilerParams(
            dimension_semantics=("arbitrary",)),
    )(entity_residual)


def kernel(text_emb, img_emb, v_missing, entity_residual, residual_scale,
           rel_emb_fusion, Wg, bg, gamma, beta, rel_emb_dec, has_img,
           pos_triples, neg_triples):
    heids = jnp.concatenate([pos_triples[:, 0], neg_triples[:, 0]])
    teids = jnp.concatenate([pos_triples[:, 2], neg_triples[:, 2]])
    rids = jnp.concatenate([pos_triples[:, 1], neg_triples[:, 1]])
    bg2 = bg.reshape(1, D)
    gamma2 = gamma.reshape(1, D)
    beta2 = beta.reshape(1, D)
    vm2 = v_missing.reshape(1, D)
    rs2 = jnp.asarray(residual_scale, jnp.float32).reshape(1, 1)
    has2d = has_img.astype(jnp.float32).reshape(N_ENT, 1)

    combo_ti, l2 = _tc_pack_ent(text_emb, img_emb, has2d, vm2,
                                entity_residual)
    combo_rel = _tc_pack(rel_emb_fusion, rel_emb_dec)

    total = l2[0, 0]
    bces = []
    for s in range(N_STRIPE):
        lo, hi = s * TRI_S, (s + 1) * TRI_S
        eids_s = jnp.concatenate([heids[lo:hi], teids[lo:hi]])
        rids_s = rids[lo:hi]
        n_pos_s = min(max(B_POS - lo, 0), TRI_S)
        assert n_pos_s % ROWS_B == 0

        if s >= 2:
            eids_s, _ = lax.optimization_barrier((eids_s, bces[s - 2]))

        gat_ti, gat_r, gat_rel = _sc_gather(
            combo_ti, entity_residual, combo_rel,
            eids_s.reshape(NW, 2 * TRI_S // (NW * CHUNK), CHUNK),
            rids_s.reshape(NW, TRI_S // (NW * CHUNK), CHUNK))

        bce_s = _tc_fuse(
            gat_ti, gat_r, gat_rel,
            Wg, bg2, gamma2, beta2, rs2,
            TRI_S, n_pos_s // ROWS_B, s == 0)
        bces.append(bce_s)
        total = total + bce_s[0, 0]

    return total
